# baseline (speedup 1.0000x reference)
import jax
import jax.numpy as jnp
from jax import lax
from jax.experimental import pallas as pl
from jax.experimental.pallas import tpu as pltpu
from jax.experimental.pallas import tpu_sc as plsc

N = 10000
E = 320000
D = 128
NC = 2
NS = 16
NW = NC * NS
EPW = E // NW
C = 128
NCH = 80
EPAD = NCH * C - EPW
PR = 5120
NP = 5632
DPT = PR // NS
DCH = 80
SEG = D // 16


def _sc_body(x_hbm, src_hbm, tgt_hbm, ew_hbm, out_hbm,
             src_v, tgt_v, ew_v, rows_a, rows_b, tloc_v, zbuf_v, delta_sh,
             sem_a, sem_b):
    cid = lax.axis_index("c")
    sid = lax.axis_index("s")
    wid = cid * NS + sid

    pltpu.sync_copy(src_hbm.at[wid], src_v)
    pltpu.sync_copy(tgt_hbm.at[wid], tgt_v)
    pltpu.sync_copy(ew_hbm.at[wid], ew_v)

    zeros16 = jnp.zeros((16,), jnp.float32)

    for p in range(2):
        lo = p * PR

        @plsc.parallel_loop(0, DCH)
        def _zero_row(r):
            for s in range(SEG):
                zbuf_v[r, pl.ds(s * 16, 16)] = zeros16

        for k in range(DPT // DCH):
            pltpu.sync_copy(zbuf_v, delta_sh.at[pl.ds(sid * DPT + k * DCH, DCH)])
        plsc.subcore_barrier()

        def process(j, rows_v, trow):
            @plsc.parallel_loop(0, C, step=16)
            def _scale(g):
                t16 = tgt_v[j, pl.ds(g, 16)] - lo
                ok = (t16 >= 0) & (t16 < PR)
                tloc_v[trow, pl.ds(g, 16)] = jnp.where(ok, t16, PR)
                w16 = ew_v[j, pl.ds(g, 16)]
                for l in range(16):
                    w = w16[l]
                    for s in range(SEG):
                        sl = pl.ds(s * 16, 16)
                        rows_v[g + l, sl] = rows_v[g + l, sl] * w

            pltpu.sync_copy(rows_v, delta_sh.at[tloc_v.at[trow]], add=True)

        def chunk_body(j, carry):
            pltpu.async_copy(x_hbm.at[src_v.at[j]], rows_a, sem_a).wait()
            process(j, rows_a, 0)
            return carry

        lax.fori_loop(0, NCH, chunk_body, 0)
        plsc.subcore_barrier()

        for k in range(DPT // DCH):
            r = sid * DPT + k * DCH

            @pl.when(lo + r < N)
            def _drain():
                pltpu.sync_copy(delta_sh.at[pl.ds(r, DCH)], zbuf_v)
                pltpu.sync_copy(zbuf_v, out_hbm.at[cid].at[pl.ds(lo + r, DCH)])

        if p == 0:
            plsc.subcore_barrier()


_sc_scatter = pl.kernel(
    _sc_body,
    out_type=jax.ShapeDtypeStruct((NC, N, D), jnp.float32),
    mesh=plsc.VectorSubcoreMesh(core_axis_name="c", subcore_axis_name="s"),
    scratch_types=[
        pltpu.VMEM((NCH, C), jnp.int32),
        pltpu.VMEM((NCH, C), jnp.int32),
        pltpu.VMEM((NCH, C), jnp.float32),
        pltpu.VMEM((C, D), jnp.float32),
        pltpu.VMEM((C, D), jnp.float32),
        pltpu.VMEM((2, C), jnp.int32),
        pltpu.VMEM((DCH, D), jnp.float32),
        pltpu.VMEM_SHARED((NP, D), jnp.float32),
        pltpu.SemaphoreType.DMA,
        pltpu.SemaphoreType.DMA,
    ],
)


BLK = 400


def _tc_body(x_ref, d0_ref, d1_ref, ws_ref, wd_ref, b_ref, o_ref):
    d = d0_ref[...] + d1_ref[...]
    o_ref[...] = (
        jnp.dot(x_ref[...], ws_ref[...], preferred_element_type=jnp.float32)
        + jnp.dot(d, wd_ref[...], preferred_element_type=jnp.float32)
        + b_ref[...]
    )


_node_update = pl.pallas_call(
    _tc_body,
    grid=(N // BLK,),
    in_specs=[
        pl.BlockSpec((BLK, D), lambda i: (i, 0)),
        pl.BlockSpec((BLK, D), lambda i: (i, 0)),
        pl.BlockSpec((BLK, D), lambda i: (i, 0)),
        pl.BlockSpec((D, D), lambda i: (0, 0)),
        pl.BlockSpec((D, D), lambda i: (0, 0)),
        pl.BlockSpec((1, D), lambda i: (0, 0)),
    ],
    out_specs=pl.BlockSpec((BLK, D), lambda i: (i, 0)),
    out_shape=jax.ShapeDtypeStruct((N, D), jnp.float32),
)


@jax.jit
def kernel(x, source, target, edge_weights, W_self, W_delta, b):
    src2 = jnp.pad(source.reshape(NW, EPW), ((0, 0), (0, EPAD)))
    tgt2 = jnp.pad(target.reshape(NW, EPW), ((0, 0), (0, EPAD)))
    ew2 = jnp.pad(edge_weights.reshape(NW, EPW), ((0, 0), (0, EPAD)))
    d = _sc_scatter(x, src2.reshape(NW, NCH, C), tgt2.reshape(NW, NCH, C),
                    ew2.reshape(NW, NCH, C))
    return _node_update(x, d[0], d[1], W_self, W_delta, b.reshape(1, D))


# speedup vs baseline: 1.9698x; 1.9698x over previous
import jax
import jax.numpy as jnp
from jax import lax
from jax.experimental import pallas as pl
from jax.experimental.pallas import tpu as pltpu
from jax.experimental.pallas import tpu_sc as plsc

N = 10000
E = 320000
D = 128
NC = 2
NS = 16
NW = NC * NS
EPW = E // NW
C = 80
NCH = 126
EPAD = NCH * C - EPW
PR = 5120
NP = 5632
DPT = PR // NS
DCH = 80
SEG = D // 16


def _sc_body(x_hbm, src_hbm, tgt_hbm, ew_hbm, out_hbm,
             src_v, tgt_v, ew_v, rows_a, rows_b, tloc_v, zbuf_v, delta_sh,
             sem_a, sem_b):
    cid = lax.axis_index("c")
    sid = lax.axis_index("s")
    wid = cid * NS + sid

    pltpu.sync_copy(src_hbm.at[wid], src_v)
    pltpu.sync_copy(tgt_hbm.at[wid], tgt_v)
    pltpu.sync_copy(ew_hbm.at[wid], ew_v)

    zeros16 = jnp.zeros((16,), jnp.float32)

    for p in range(2):
        lo = p * PR

        @plsc.parallel_loop(0, DCH)
        def _zero_row(r):
            for s in range(SEG):
                zbuf_v[r, pl.ds(s * 16, 16)] = zeros16

        for k in range(DPT // DCH):
            pltpu.sync_copy(zbuf_v, delta_sh.at[pl.ds(sid * DPT + k * DCH, DCH)])
        plsc.subcore_barrier()

        def process(j, rows_v, trow):
            @plsc.parallel_loop(0, C, step=16)
            def _scale(g):
                t16 = tgt_v[j, pl.ds(g, 16)] - lo
                ok = (t16 >= 0) & (t16 < PR)
                tloc_v[trow, pl.ds(g, 16)] = jnp.where(ok, t16, PR)
                w16 = ew_v[j, pl.ds(g, 16)]
                for l in range(16):
                    w = w16[l]
                    for s in range(SEG):
                        sl = pl.ds(s * 16, 16)
                        rows_v[g + l, sl] = rows_v[g + l, sl] * w

            pltpu.sync_copy(rows_v, delta_sh.at[tloc_v.at[trow]], add=True)

        pltpu.async_copy(x_hbm.at[src_v.at[0]], rows_a, sem_a)

        def chunk_pair(j2, carry):
            ja = 2 * j2
            pltpu.async_copy(x_hbm.at[src_v.at[ja + 1]], rows_b, sem_b)
            pltpu.make_async_copy(x_hbm.at[src_v.at[ja]], rows_a, sem_a).wait()
            process(ja, rows_a, 0)

            @pl.when(ja + 2 < NCH)
            def _next():
                pltpu.async_copy(x_hbm.at[src_v.at[ja + 2]], rows_a, sem_a)

            pltpu.make_async_copy(x_hbm.at[src_v.at[ja + 1]], rows_b,
                                  sem_b).wait()
            process(ja + 1, rows_b, 1)
            return carry

        lax.fori_loop(0, NCH // 2, chunk_pair, 0)
        plsc.subcore_barrier()

        for k in range(DPT // DCH):
            r = sid * DPT + k * DCH

            @pl.when(lo + r < N)
            def _drain():
                pltpu.sync_copy(delta_sh.at[pl.ds(r, DCH)], zbuf_v)
                pltpu.sync_copy(zbuf_v, out_hbm.at[cid].at[pl.ds(lo + r, DCH)])

        if p == 0:
            plsc.subcore_barrier()


_sc_scatter = pl.kernel(
    _sc_body,
    out_type=jax.ShapeDtypeStruct((NC, N, D), jnp.float32),
    mesh=plsc.VectorSubcoreMesh(core_axis_name="c", subcore_axis_name="s"),
    scratch_types=[
        pltpu.VMEM((NCH, C), jnp.int32),
        pltpu.VMEM((NCH, C), jnp.int32),
        pltpu.VMEM((NCH, C), jnp.float32),
        pltpu.VMEM((C, D), jnp.float32),
        pltpu.VMEM((C, D), jnp.float32),
        pltpu.VMEM((2, C), jnp.int32),
        pltpu.VMEM((DCH, D), jnp.float32),
        pltpu.VMEM_SHARED((NP, D), jnp.float32),
        pltpu.SemaphoreType.DMA,
        pltpu.SemaphoreType.DMA,
    ],
)


BLK = 400


def _tc_body(x_ref, d0_ref, d1_ref, ws_ref, wd_ref, b_ref, o_ref):
    d = d0_ref[...] + d1_ref[...]
    o_ref[...] = (
        jnp.dot(x_ref[...], ws_ref[...], preferred_element_type=jnp.float32)
        + jnp.dot(d, wd_ref[...], preferred_element_type=jnp.float32)
        + b_ref[...]
    )


_node_update = pl.pallas_call(
    _tc_body,
    grid=(N // BLK,),
    in_specs=[
        pl.BlockSpec((BLK, D), lambda i: (i, 0)),
        pl.BlockSpec((BLK, D), lambda i: (i, 0)),
        pl.BlockSpec((BLK, D), lambda i: (i, 0)),
        pl.BlockSpec((D, D), lambda i: (0, 0)),
        pl.BlockSpec((D, D), lambda i: (0, 0)),
        pl.BlockSpec((1, D), lambda i: (0, 0)),
    ],
    out_specs=pl.BlockSpec((BLK, D), lambda i: (i, 0)),
    out_shape=jax.ShapeDtypeStruct((N, D), jnp.float32),
)


@jax.jit
def kernel(x, source, target, edge_weights, W_self, W_delta, b):
    src2 = jnp.pad(source.reshape(NW, EPW), ((0, 0), (0, EPAD)))
    tgt2 = jnp.pad(target.reshape(NW, EPW), ((0, 0), (0, EPAD)))
    ew2 = jnp.pad(edge_weights.reshape(NW, EPW), ((0, 0), (0, EPAD)))
    d = _sc_scatter(x, src2.reshape(NW, NCH, C), tgt2.reshape(NW, NCH, C),
                    ew2.reshape(NW, NCH, C))
    return _node_update(x, d[0], d[1], W_self, W_delta, b.reshape(1, D))
